# baseline (speedup 1.0000x reference)
import functools

import jax
import jax.numpy as jnp
from jax import lax
from jax.experimental import pallas as pl
from jax.experimental.pallas import tpu as pltpu
from jax.experimental.pallas import tpu_sc as plsc

N = 12288
NSRC = 3
SRC_ROWS = 4096
D = 1024
P = 512
Q = N // P
NB = 128

W = 32
RPW = SRC_ROWS // W
CH = 32
NCH = RPW // CH
NIT = NSRC * NCH


def _rank_kernel(vrow_ref, out_ref, f2t_ref, macc_ref, gacc_ref, c2_ref):
    iota_b_bP = lax.broadcasted_iota(jnp.int32, (NB, P), 0)
    iota_l_Pb = lax.broadcasted_iota(jnp.int32, (P, NB), 1)
    iota_p_PP = lax.broadcasted_iota(jnp.int32, (P, P), 0)
    iota_j_PP = lax.broadcasted_iota(jnp.int32, (P, P), 1)
    diag_PP = (iota_p_PP == iota_j_PP).astype(jnp.float32)

    macc_ref[...] = jnp.zeros((NB, NB), jnp.float32)

    def loop1(q, carry):
        vrow = vrow_ref[pl.ds(q * P, P)].reshape(1, P)
        vf = vrow.astype(jnp.float32)
        vcol = jnp.sum(vf * diag_PP, axis=1, keepdims=True).astype(jnp.int32)
        l_row = jnp.bitwise_and(vrow, 127)
        h_col = jnp.right_shift(vcol, 7)
        olt = (l_row == iota_b_bP).astype(jnp.bfloat16)
        oh = (h_col == iota_l_Pb).astype(jnp.bfloat16)
        f2t = lax.dot_general(olt, oh, (((1,), (0,)), ((), ())),
                              preferred_element_type=jnp.float32)
        macc_ref[...] += f2t
        f2t_ref[pl.ds(q * NB, NB), :] = f2t.astype(jnp.bfloat16)
        eq = (vcol == vrow) & (iota_j_PP < iota_p_PP)
        c2_col = jnp.sum(jnp.where(eq, 1.0, 0.0), axis=1, keepdims=True)
        c2_row = jnp.sum(c2_col * diag_PP, axis=0, keepdims=True)
        c2_ref[pl.ds(q, 1), :] = c2_row
        return carry

    lax.fori_loop(0, Q, loop1, 0, unroll=24)

    macct = macc_ref[...]
    cnt_h = jnp.sum(macct, axis=0, keepdims=True)
    iota_bb0 = lax.broadcasted_iota(jnp.int32, (NB, NB), 0)
    iota_bb1 = lax.broadcasted_iota(jnp.int32, (NB, NB), 1)
    lt_bb = (iota_bb0 < iota_bb1).astype(jnp.float32)
    diag_bb = (iota_bb0 == iota_bb1).astype(jnp.float32)
    cnt_col = jnp.sum(cnt_h * diag_bb, axis=1, keepdims=True)
    sh_row = jnp.sum(cnt_col * lt_bb, axis=0, keepdims=True)
    m_hi = jnp.floor(macct * (1.0 / 256.0)) * 256.0
    m_lo = macct - m_hi
    m_hi_bf = m_hi.astype(jnp.bfloat16)
    m_lo_bf = m_lo.astype(jnp.bfloat16)
    sh_hi = jnp.floor(sh_row * (1.0 / 256.0)) * 256.0
    sh_lo = sh_row - sh_hi
    sh_rows = jnp.concatenate(
        [sh_hi.astype(jnp.bfloat16), sh_lo.astype(jnp.bfloat16),
         jnp.zeros((6, NB), jnp.bfloat16)], axis=0)

    gacc_ref[...] = jnp.zeros((NB, NB), jnp.float32)

    def loop2(q, carry):
        vrow = vrow_ref[pl.ds(q * P, P)].reshape(1, P)
        h_row = jnp.right_shift(vrow, 7)
        l_row = jnp.bitwise_and(vrow, 127)
        oht_bf = (h_row == iota_b_bP).astype(jnp.bfloat16)
        g = gacc_ref[...]
        g_hi = jnp.floor(g * (1.0 / 256.0)) * 256.0
        g_lo = g - g_hi
        tblt = jnp.concatenate(
            [m_hi_bf, m_lo_bf,
             g_hi.astype(jnp.bfloat16), g_lo.astype(jnp.bfloat16),
             sh_rows], axis=0)
        xyt = lax.dot_general(tblt, oht_bf, (((1,), (0,)), ((), ())),
                              preferred_element_type=jnp.float32)
        xt = xyt[0:NB, :] + xyt[NB:2 * NB, :]
        yt = xyt[2 * NB:3 * NB, :] + xyt[3 * NB:4 * NB, :]
        a_row = xyt[4 * NB:4 * NB + 1, :] + xyt[4 * NB + 1:4 * NB + 2, :]
        b_row = jnp.sum(jnp.where(iota_b_bP < l_row, xt, 0.0), axis=0,
                        keepdims=True)
        c1_row = jnp.sum(jnp.where(iota_b_bP == l_row, yt, 0.0), axis=0,
                         keepdims=True)
        rank = a_row + b_row + c1_row + c2_ref[pl.ds(q, 1), :]
        out_ref[pl.ds(q, 1), :] = rank.astype(jnp.int32)
        gacc_ref[...] += f2t_ref[pl.ds(q * NB, NB), :].astype(jnp.float32)
        return carry

    lax.fori_loop(0, Q, loop2, 0, unroll=24)


def _compute_rank(vrow):
    return pl.pallas_call(
        _rank_kernel,
        out_shape=jax.ShapeDtypeStruct((Q, P), jnp.int32),
        scratch_shapes=[
            pltpu.VMEM((Q * NB, NB), jnp.bfloat16),
            pltpu.VMEM((NB, NB), jnp.float32),
            pltpu.VMEM((NB, NB), jnp.float32),
            pltpu.VMEM((Q, P), jnp.float32),
        ],
    )(vrow)


NBUF = 3


def _sc_scatter(asc, cru, des, rank):
    mesh = plsc.VectorSubcoreMesh(core_axis_name="c", subcore_axis_name="s")

    @functools.partial(
        pl.kernel,
        out_type=jax.ShapeDtypeStruct((N, D), jnp.float32),
        mesh=mesh,
        scratch_types=(
            [pltpu.VMEM((NIT, CH), jnp.int32)]
            + [pltpu.VMEM((CH, D), jnp.float32)] * NBUF
            + [pltpu.SemaphoreType.DMA] * (2 * NBUF)
        ),
    )
    def scatter_kernel(asc_hbm, cru_hbm, des_hbm, rank_hbm, out_hbm,
                       idx_v, *bufs_sems):
        bufs = bufs_sems[:NBUF]
        lsems = bufs_sems[NBUF:2 * NBUF]
        ssems = bufs_sems[2 * NBUF:3 * NBUF]
        wid = lax.axis_index("s") * 2 + lax.axis_index("c")
        row0 = wid * RPW
        srcs = (asc_hbm, cru_hbm, des_hbm)
        for s in range(NSRC):
            pltpu.sync_copy(
                rank_hbm.at[pl.ds(s * (SRC_ROWS // CH) + wid * NCH, NCH)],
                idx_v.at[pl.ds(s * NCH, NCH)])

        def start_load(i):
            s, k = divmod(i, NCH)
            return pltpu.async_copy(
                srcs[s].at[pl.ds(row0 + k * CH, CH)], bufs[i % NBUF],
                lsems[i % NBUF])

        loads = {i: start_load(i) for i in range(min(2, NIT))}
        stores = {}
        for i in range(NIT):
            loads[i].wait()
            stores[i] = pltpu.async_copy(
                bufs[i % NBUF], out_hbm.at[idx_v.at[i]], ssems[i % NBUF])
            m = i + 2
            if m < NIT:
                if m - NBUF >= 0:
                    stores[m - NBUF].wait()
                loads[m] = start_load(m)
        for i in range(NIT - NBUF, NIT):
            stores[i].wait()

    return scatter_kernel(asc, cru, des, rank)


def kernel(asc_dec, cru_dec, des_dec, concat_index):
    v = concat_index.astype(jnp.int32)
    rank = _compute_rank(v)
    rank2d = rank.reshape(N // CH, CH)
    return _sc_scatter(asc_dec, cru_dec, des_dec, rank2d)
